# baseline (speedup 1.0000x reference)
import functools

import jax
import jax.numpy as jnp
from jax.experimental import pallas as pl
from jax.experimental.pallas import tpu as pltpu

_EPS = 1e-12
_LOG2E = 1.4426950408889634


def _body(q_ref, m_ref, s0_ref, s1_ref, s2_ref, s3_ref, rb_ref, v_ref,
          qb_vmem, sum_vmem, acc_vmem, *, tile_n, n_modes, n_tiles, n_pad,
          stripe):
    j = pl.program_id(0)

    @pl.when(j == 0)
    def _prologue():
        q = q_ref[...]
        qn_inv = 1.0 / jnp.maximum(
            jnp.sqrt(jnp.sum(q * q, axis=1, keepdims=True)), _EPS)
        qb_vmem[...] = (q * qn_inv).astype(jnp.bfloat16)
        sum_vmem[...] = jnp.zeros_like(sum_vmem)
        acc_vmem[...] = jnp.zeros_like(acc_vmem)

    m = m_ref[...]
    row = j * tile_n + jax.lax.broadcasted_iota(jnp.int32, m.shape, 0)
    m = jnp.where(row < n_modes, m, 0.0)
    mn_inv2 = _LOG2E / jnp.maximum(jnp.sqrt(jnp.sum(m * m, axis=1)), _EPS)
    ms = (m * mn_inv2[:, None]).astype(jnp.bfloat16)
    s2 = jax.lax.dot_general(qb_vmem[...], ms, (((1,), (1,)), ((), ())),
                             preferred_element_type=jnp.float32)
    s2b = s2
    s0_ref[...] = s2b[0 * stripe:1 * stripe]
    s1_ref[...] = s2b[1 * stripe:2 * stripe]
    s2_ref[...] = s2b[2 * stripe:3 * stripe]
    s3_ref[...] = s2b[3 * stripe:4 * stripe]
    e = jnp.exp2(s2)
    sum_vmem[...] += jnp.sum(e, axis=1, keepdims=True)
    acc_vmem[...] += jax.lax.dot_general(e.astype(jnp.bfloat16),
                                         m.astype(jnp.bfloat16),
                                         (((1,), (0,)), ((), ())),
                                         preferred_element_type=jnp.float32)

    @pl.when(j == n_tiles - 1)
    def _epilogue():
        sumexp = sum_vmem[...] - float(n_pad - n_modes)
        rb_ref[...] = -jnp.log2(sumexp)
        v_ref[...] = acc_vmem[...] / sumexp


@functools.partial(jax.jit, static_argnames=("tile_n",))
def _run(query, modes, tile_n=2048):
    b, d = query.shape
    n = modes.shape[0]
    n_tiles = pl.cdiv(n, tile_n)
    n_pad = n_tiles * tile_n
    stripe = b // 4

    stripe_spec = pl.BlockSpec((stripe, tile_n), lambda j: (0, j))
    stripe_shape = jax.ShapeDtypeStruct((stripe, n), jnp.float32)

    s0, s1, s2, s3, rowbias, value = pl.pallas_call(
        functools.partial(_body, tile_n=tile_n, n_modes=n, n_tiles=n_tiles,
                          n_pad=n_pad, stripe=stripe),
        grid=(n_tiles,),
        in_specs=[pl.BlockSpec((b, d), lambda j: (0, 0)),
                  pl.BlockSpec((tile_n, d), lambda j: (j, 0))],
        out_specs=[stripe_spec, stripe_spec, stripe_spec, stripe_spec,
                   pl.BlockSpec((b, 1), lambda j: (0, 0)),
                   pl.BlockSpec((b, d), lambda j: (0, 0))],
        out_shape=[stripe_shape, stripe_shape, stripe_shape, stripe_shape,
                   jax.ShapeDtypeStruct((b, 1), jnp.float32),
                   jax.ShapeDtypeStruct((b, d), jnp.float32)],
        scratch_shapes=[pltpu.VMEM((b, d), jnp.bfloat16),
                        pltpu.VMEM((b, 1), jnp.float32),
                        pltpu.VMEM((b, d), jnp.float32)],
        compiler_params=pltpu.CompilerParams(
            dimension_semantics=("arbitrary",)),
    )(query, modes)

    s2_full = jnp.concatenate([s0, s1, s2, s3], axis=0)
    weights = jnp.exp2(s2_full + rowbias)
    return value, weights


def kernel(query, modes):
    return _run(query, modes)


# speedup vs baseline: 1.1112x; 1.1112x over previous
import functools

import jax
import jax.numpy as jnp
from jax.experimental import pallas as pl
from jax.experimental.pallas import tpu as pltpu

_EPS = 1e-12
_LOG2E = 1.4426950408889634


def _body(q_ref, m_ref, s0_ref, rb_ref, v_ref,
          qb_vmem, sum_vmem, acc_vmem, *, tile_n, n_modes, n_tiles, n_pad,
          stripe):
    j = pl.program_id(0)

    @pl.when(j == 0)
    def _prologue():
        q = q_ref[...]
        qn_inv = 1.0 / jnp.maximum(
            jnp.sqrt(jnp.sum(q * q, axis=1, keepdims=True)), _EPS)
        qb_vmem[...] = (q * qn_inv).astype(jnp.bfloat16)
        sum_vmem[...] = jnp.zeros_like(sum_vmem)
        acc_vmem[...] = jnp.zeros_like(acc_vmem)

    m = m_ref[...]
    row = j * tile_n + jax.lax.broadcasted_iota(jnp.int32, m.shape, 0)
    m = jnp.where(row < n_modes, m, 0.0)
    mn_inv2 = _LOG2E / jnp.maximum(jnp.sqrt(jnp.sum(m * m, axis=1)), _EPS)
    ms = (m * mn_inv2[:, None]).astype(jnp.bfloat16)
    s2 = jax.lax.dot_general(qb_vmem[...], ms, (((1,), (1,)), ((), ())),
                             preferred_element_type=jnp.float32)
    s0_ref[...] = s2.astype(jnp.bfloat16)
    e = jnp.exp2(s2)
    sum_vmem[...] += jnp.sum(e, axis=1, keepdims=True)
    acc_vmem[...] += jax.lax.dot_general(e.astype(jnp.bfloat16),
                                         m.astype(jnp.bfloat16),
                                         (((1,), (0,)), ((), ())),
                                         preferred_element_type=jnp.float32)

    @pl.when(j == n_tiles - 1)
    def _epilogue():
        sumexp = sum_vmem[...] - float(n_pad - n_modes)
        rb_ref[...] = -jnp.log2(sumexp)
        v_ref[...] = acc_vmem[...] / sumexp


@functools.partial(jax.jit, static_argnames=("tile_n",))
def _run(query, modes, tile_n=2048):
    b, d = query.shape
    n = modes.shape[0]
    n_tiles = pl.cdiv(n, tile_n)
    n_pad = n_tiles * tile_n
    stripe = b // 4

    stripe_spec = pl.BlockSpec((b, tile_n), lambda j: (0, j))
    stripe_shape = jax.ShapeDtypeStruct((b, n), jnp.bfloat16)

    s0, rowbias, value = pl.pallas_call(
        functools.partial(_body, tile_n=tile_n, n_modes=n, n_tiles=n_tiles,
                          n_pad=n_pad, stripe=stripe),
        grid=(n_tiles,),
        in_specs=[pl.BlockSpec((b, d), lambda j: (0, 0)),
                  pl.BlockSpec((tile_n, d), lambda j: (j, 0))],
        out_specs=[stripe_spec,
                   pl.BlockSpec((b, 1), lambda j: (0, 0)),
                   pl.BlockSpec((b, d), lambda j: (0, 0))],
        out_shape=[stripe_shape,
                   jax.ShapeDtypeStruct((b, 1), jnp.float32),
                   jax.ShapeDtypeStruct((b, d), jnp.float32)],
        scratch_shapes=[pltpu.VMEM((b, d), jnp.bfloat16),
                        pltpu.VMEM((b, 1), jnp.float32),
                        pltpu.VMEM((b, d), jnp.float32)],
        compiler_params=pltpu.CompilerParams(
            dimension_semantics=("arbitrary",)),
    )(query, modes)

    weights = jnp.exp2(s0.astype(jnp.float32) + rowbias)
    return value, weights


def kernel(query, modes):
    return _run(query, modes)


# speedup vs baseline: 1.2151x; 1.0935x over previous
import functools

import jax
import jax.numpy as jnp
from jax.experimental import pallas as pl
from jax.experimental.pallas import tpu as pltpu

_EPS = 1e-12
_LOG2E = 1.4426950408889634


def _sum_body(q_ref, m_ref, qb_ref, ms_ref, mr_ref, rb_ref, qb_vmem, acc_ref,
              *, tile_n, n_modes, n_tiles, n_pad):
    j = pl.program_id(0)

    @pl.when(j == 0)
    def _prologue():
        q = q_ref[...]
        qn_inv = 1.0 / jnp.maximum(
            jnp.sqrt(jnp.sum(q * q, axis=1, keepdims=True)), _EPS)
        qb = (q * qn_inv).astype(jnp.bfloat16)
        qb_vmem[...] = qb
        qb_ref[...] = qb
        acc_ref[...] = jnp.zeros_like(acc_ref)

    m = m_ref[...]
    row = j * tile_n + jax.lax.broadcasted_iota(jnp.int32, m.shape, 0)
    m = jnp.where(row < n_modes, m, 0.0)
    mn_inv2 = _LOG2E / jnp.maximum(jnp.sqrt(jnp.sum(m * m, axis=1)), _EPS)
    ms = (m * mn_inv2[:, None]).astype(jnp.bfloat16)
    ms_ref[...] = ms
    mr_ref[...] = m.astype(jnp.bfloat16)
    s2 = jax.lax.dot_general(qb_vmem[...], ms, (((1,), (1,)), ((), ())),
                             preferred_element_type=jnp.float32)
    acc_ref[...] += jnp.sum(jnp.exp2(s2), axis=1, keepdims=True)

    @pl.when(j == n_tiles - 1)
    def _epilogue():
        sumexp = acc_ref[...] - float(n_pad - n_modes)
        rb_ref[...] = -jnp.log2(sumexp)


def _write_body(qb_ref, ms_ref, mr_ref, rb_ref, w_ref, v_ref, acc_ref, *,
                n_tiles):
    j = pl.program_id(0)
    s2 = jax.lax.dot_general(qb_ref[...], ms_ref[...], (((1,), (1,)), ((), ())),
                             preferred_element_type=jnp.float32)
    w = jnp.exp2(s2 + rb_ref[...])
    w_ref[...] = w

    @pl.when(j == 0)
    def _init():
        acc_ref[...] = jnp.zeros_like(acc_ref)

    acc_ref[...] += jax.lax.dot_general(w.astype(jnp.bfloat16), mr_ref[...],
                                        (((1,), (0,)), ((), ())),
                                        preferred_element_type=jnp.float32)

    @pl.when(j == n_tiles - 1)
    def _fin():
        v_ref[...] = acc_ref[...]


@functools.partial(jax.jit, static_argnames=("tile_n",))
def _run(query, modes, tile_n=2048):
    b, d = query.shape
    n = modes.shape[0]
    n_tiles = pl.cdiv(n, tile_n)
    n_pad = n_tiles * tile_n

    vec_spec = pl.BlockSpec((b, 1), lambda j: (0, 0))
    q_spec = pl.BlockSpec((b, d), lambda j: (0, 0))
    m_spec = pl.BlockSpec((tile_n, d), lambda j: (j, 0))

    qb, mscaled, mraw, rowbias = pl.pallas_call(
        functools.partial(_sum_body, tile_n=tile_n, n_modes=n,
                          n_tiles=n_tiles, n_pad=n_pad),
        grid=(n_tiles,),
        in_specs=[q_spec, m_spec],
        out_specs=[q_spec, m_spec, m_spec, vec_spec],
        out_shape=[jax.ShapeDtypeStruct((b, d), jnp.bfloat16),
                   jax.ShapeDtypeStruct((n_pad, d), jnp.bfloat16),
                   jax.ShapeDtypeStruct((n_pad, d), jnp.bfloat16),
                   jax.ShapeDtypeStruct((b, 1), jnp.float32)],
        scratch_shapes=[pltpu.VMEM((b, d), jnp.bfloat16),
                        pltpu.VMEM((b, 1), jnp.float32)],
        compiler_params=pltpu.CompilerParams(
            dimension_semantics=("arbitrary",)),
    )(query, modes)

    weights, value = pl.pallas_call(
        functools.partial(_write_body, n_tiles=n_tiles),
        grid=(n_tiles,),
        in_specs=[q_spec, m_spec, m_spec, vec_spec],
        out_specs=[pl.BlockSpec((b, tile_n), lambda j: (0, j)),
                   pl.BlockSpec((b, d), lambda j: (0, 0))],
        out_shape=[jax.ShapeDtypeStruct((b, n), jnp.float32),
                   jax.ShapeDtypeStruct((b, d), jnp.float32)],
        scratch_shapes=[pltpu.VMEM((b, d), jnp.float32)],
        compiler_params=pltpu.CompilerParams(
            dimension_semantics=("arbitrary",)),
    )(qb, mscaled, mraw, rowbias)

    return value, weights


def kernel(query, modes):
    return _run(query, modes)
